# Initial kernel scaffold; baseline (speedup 1.0000x reference)
import jax
import jax.numpy as jnp
from jax.experimental import pallas as pl


def kernel(x, edge_index, W1, b1, W2, b2, Wl, bl):
    raise NotImplementedError("write your pallas kernel here")


# speedup vs baseline: 13.2980x; 13.2980x over previous
import functools

import jax
import jax.numpy as jnp
from jax import lax
from jax.experimental import pallas as pl
from jax.experimental.pallas import tpu as pltpu
from jax.experimental.pallas import tpu_sc as plsc

NC = 2
NS = 16
NW = NC * NS
CH = 128
DPAD1 = 16
DPAD2 = 8

F32 = jnp.float32


def _leaky(v):
    return jnp.where(v >= 0, v, 0.01 * v)


def _row_chunk(rpt):
    best = 8
    for czc in range(8, min(rpt, 512) + 1, 8):
        if rpt % czc == 0:
            best = czc
    return best


def _sc_mesh():
    return plsc.VectorSubcoreMesh(core_axis_name="c", subcore_axis_name="s",
                                  num_cores=NC, num_subcores=NS)


_SC_PARAMS = pltpu.CompilerParams(use_tc_tiling_on_sc=False)


@functools.partial(jax.jit, static_argnames=("np_", "nc_"))
def _sc_degree(dst3, zeros_r, np_, nc_):
    rpt = np_ // NS

    def body(dst_hbm, zer_hbm, out_hbm, dst_v, ones_v, buf_v, acc_sh, sem):
        c = lax.axis_index("c")
        s = lax.axis_index("s")
        w = c * NS + s
        base = pl.multiple_of(s * rpt, 8)
        pltpu.sync_copy(zer_hbm, buf_v)
        pltpu.sync_copy(buf_v, acc_sh.at[pl.ds(base, rpt)])
        for i in range(CH // 16):
            ones_v[pl.ds(16 * i, 16)] = jnp.full((16,), 1.0, F32)
        plsc.subcore_barrier()

        @pl.loop(0, nc_)
        def _(j):
            pltpu.sync_copy(dst_hbm.at[w, j], dst_v)
            pltpu.sync_copy(ones_v, acc_sh.at[dst_v], add=True)

        plsc.subcore_barrier()
        obase = pl.multiple_of(c * np_ + s * rpt, 8)
        pltpu.sync_copy(acc_sh.at[pl.ds(base, rpt)], buf_v)
        pltpu.sync_copy(buf_v, out_hbm.at[pl.ds(obase, rpt)])

    return pl.kernel(
        body,
        out_type=jax.ShapeDtypeStruct((NC * np_,), F32),
        mesh=_sc_mesh(),
        compiler_params=_SC_PARAMS,
        scratch_types=[
            pltpu.VMEM((CH,), jnp.int32),
            pltpu.VMEM((CH,), F32),
            pltpu.VMEM((rpt,), F32),
            pltpu.VMEM_SHARED((np_,), F32),
            pltpu.SemaphoreType.DMA,
        ],
    )(dst3, zeros_r)


@functools.partial(jax.jit, static_argnames=("np_", "nc_", "d", "colsplit"))
def _sc_aggregate(src3, dst3, table, zeros_cd, np_, nc_, d, colsplit):
    rpt = np_ // NS
    cz = _row_chunk(rpt)

    def body(src_hbm, dst_hbm, tab_hbm, zer_hbm, out_hbm,
             src_v, dst_v, rows_v, buf_v, acc_sh, sem):
        c = lax.axis_index("c")
        s = lax.axis_index("s")
        pltpu.sync_copy(zer_hbm, buf_v)

        @pl.loop(0, rpt // cz)
        def _(k):
            zb = pl.multiple_of(s * rpt + k * cz, 8)
            pltpu.sync_copy(buf_v, acc_sh.at[pl.ds(zb, cz)])

        plsc.subcore_barrier()

        if colsplit:
            @pl.loop(0, nc_)
            def _(j):
                pltpu.sync_copy(src_hbm.at[s, j], src_v)
                pltpu.sync_copy(dst_hbm.at[s, j], dst_v)
                pltpu.async_copy(tab_hbm.at[c].at[src_v], rows_v, sem).wait()
                pltpu.sync_copy(rows_v, acc_sh.at[dst_v], add=True)
        else:
            w = c * NS + s

            @pl.loop(0, nc_)
            def _(j):
                pltpu.sync_copy(src_hbm.at[w, j], src_v)
                pltpu.sync_copy(dst_hbm.at[w, j], dst_v)
                pltpu.async_copy(tab_hbm.at[src_v], rows_v, sem).wait()
                pltpu.sync_copy(rows_v, acc_sh.at[dst_v], add=True)

        plsc.subcore_barrier()

        @pl.loop(0, rpt // cz)
        def _(k):
            ib = pl.multiple_of(s * rpt + k * cz, 8)
            ob = pl.multiple_of(c * np_ + s * rpt + k * cz, 8)
            pltpu.sync_copy(acc_sh.at[pl.ds(ib, cz)], buf_v)
            pltpu.sync_copy(buf_v, out_hbm.at[pl.ds(ob, cz)])

    return pl.kernel(
        body,
        out_type=jax.ShapeDtypeStruct((NC * np_, d), F32),
        mesh=_sc_mesh(),
        compiler_params=_SC_PARAMS,
        scratch_types=[
            pltpu.VMEM((CH,), jnp.int32),
            pltpu.VMEM((CH,), jnp.int32),
            pltpu.VMEM((CH, d), F32),
            pltpu.VMEM((cz, d), F32),
            pltpu.VMEM_SHARED((np_, d), F32),
            pltpu.SemaphoreType.DMA,
        ],
    )(src3, dst3, table, zeros_cd)


def _tc1(xp, dega, degb, w1p, bn, np_):
    fin = xp.shape[1]
    fo = w1p.shape[1]
    half = fo // 2

    def body(x_ref, da_ref, db_ref, w_ref, h_ref, hn_ref, di_ref):
        deg = da_ref[...] + db_ref[...] + 1.0
        dinv = lax.rsqrt(deg)
        h = jnp.dot(x_ref[...], w_ref[...], preferred_element_type=F32)
        h_ref[...] = h
        hn = h * dinv
        zpad = jnp.zeros((h.shape[0], DPAD1 - half), F32)
        hn_ref[0] = jnp.concatenate([hn[:, :half], zpad], axis=1)
        hn_ref[1] = jnp.concatenate([hn[:, half:], zpad], axis=1)
        di_ref[...] = dinv

    return pl.pallas_call(
        body,
        grid=(np_ // bn,),
        in_specs=[
            pl.BlockSpec((bn, fin), lambda i: (i, 0)),
            pl.BlockSpec((bn, 1), lambda i: (i, 0)),
            pl.BlockSpec((bn, 1), lambda i: (i, 0)),
            pl.BlockSpec((fin, fo), lambda i: (0, 0)),
        ],
        out_specs=[
            pl.BlockSpec((bn, fo), lambda i: (i, 0)),
            pl.BlockSpec((NC, bn, DPAD1), lambda i: (0, i, 0)),
            pl.BlockSpec((bn, 1), lambda i: (i, 0)),
        ],
        out_shape=[
            jax.ShapeDtypeStruct((np_, fo), F32),
            jax.ShapeDtypeStruct((NC, np_, DPAD1), F32),
            jax.ShapeDtypeStruct((np_, 1), F32),
        ],
    )(xp, dega, degb, w1p)


def _tc2(acca, accb, h1, dinv, b1p, w2, bn, np_):
    fo = h1.shape[1]
    half = fo // 2
    f3 = w2.shape[1]

    def body(aa_ref, ab_ref, h_ref, di_ref, b_ref, w_ref, h3_ref, hn3_ref):
        dinv_c = di_ref[...]
        agg = jnp.concatenate([aa_ref[...][:, :half], ab_ref[...][:, :half]],
                              axis=1)
        out1 = dinv_c * agg + (dinv_c * dinv_c) * h_ref[...] + b_ref[...]
        out1 = _leaky(out1)
        h2 = jnp.maximum(out1[:, :half], out1[:, half:])
        h3 = jnp.dot(h2, w_ref[...], preferred_element_type=F32)
        h3_ref[...] = h3
        hn3 = h3 * dinv_c
        zpad = jnp.zeros((h3.shape[0], DPAD2 - f3), F32)
        hn3_ref[...] = jnp.concatenate([hn3, zpad], axis=1)

    return pl.pallas_call(
        body,
        grid=(np_ // bn,),
        in_specs=[
            pl.BlockSpec((bn, DPAD1), lambda i: (i, 0)),
            pl.BlockSpec((bn, DPAD1), lambda i: (i, 0)),
            pl.BlockSpec((bn, fo), lambda i: (i, 0)),
            pl.BlockSpec((bn, 1), lambda i: (i, 0)),
            pl.BlockSpec((1, fo), lambda i: (0, 0)),
            pl.BlockSpec((half, f3), lambda i: (0, 0)),
        ],
        out_specs=[
            pl.BlockSpec((bn, f3), lambda i: (i, 0)),
            pl.BlockSpec((bn, DPAD2), lambda i: (i, 0)),
        ],
        out_shape=[
            jax.ShapeDtypeStruct((np_, f3), F32),
            jax.ShapeDtypeStruct((np_, DPAD2), F32),
        ],
    )(acca, accb, h1, dinv, b1p, w2)


def _tc3(acca, accb, h3, dinv, b2, wl, bl, bn, np_):
    f3 = h3.shape[1]
    fl = wl.shape[1]

    def body(aa_ref, ab_ref, h_ref, di_ref, b2_ref, w_ref, bl_ref, o_ref):
        dinv_c = di_ref[...]
        agg = aa_ref[...][:, :f3] + ab_ref[...][:, :f3]
        out2 = dinv_c * agg + (dinv_c * dinv_c) * h_ref[...] + b2_ref[...]
        out2 = _leaky(out2)
        o_ref[...] = (jnp.dot(out2, w_ref[...], preferred_element_type=F32)
                      + bl_ref[...])

    return pl.pallas_call(
        body,
        grid=(np_ // bn,),
        in_specs=[
            pl.BlockSpec((bn, DPAD2), lambda i: (i, 0)),
            pl.BlockSpec((bn, DPAD2), lambda i: (i, 0)),
            pl.BlockSpec((bn, f3), lambda i: (i, 0)),
            pl.BlockSpec((bn, 1), lambda i: (i, 0)),
            pl.BlockSpec((1, f3), lambda i: (0, 0)),
            pl.BlockSpec((f3, fl), lambda i: (0, 0)),
            pl.BlockSpec((1, fl), lambda i: (0, 0)),
        ],
        out_specs=pl.BlockSpec((bn, fl), lambda i: (i, 0)),
        out_shape=jax.ShapeDtypeStruct((np_, fl), F32),
    )(acca, accb, h3, dinv, b2, wl, bl)


def kernel(x, edge_index, W1, b1, W2, b2, Wl, bl):
    n = x.shape[0]
    e = edge_index.shape[1]
    fo = W1.shape[1]
    half = fo // 2
    f3 = W2.shape[1]

    rpt = -(-(n + 1) // (NS * 8)) * 8
    np_ = rpt * NS
    nc_ = -(-e // (NW * CH))
    ep = NW * CH * nc_

    perm = jnp.arange(fo).reshape(fo // 2, 2).T.reshape(fo)
    w1p = W1[:, perm]
    b1p = b1[perm][None, :]

    src_f = jnp.concatenate(
        [edge_index[0], jnp.full((ep - e,), n, jnp.int32)])
    dst_f = jnp.concatenate(
        [edge_index[1], jnp.full((ep - e,), n, jnp.int32)])
    src2 = src_f.reshape(NW, nc_, CH)
    dst2 = dst_f.reshape(NW, nc_, CH)
    src1 = src_f.reshape(NS, NC * nc_, CH)
    dst1 = dst_f.reshape(NS, NC * nc_, CH)
    xp = jnp.concatenate(
        [x, jnp.zeros((np_ - n, x.shape[1]), F32)], axis=0)

    cz = _row_chunk(rpt)
    zer_r = jnp.zeros((rpt,), F32)
    zer_ch = jnp.zeros((cz, DPAD1), F32)
    zer_cf = jnp.zeros((cz, DPAD2), F32)

    bn = np_ // 16

    deg = _sc_degree(dst2, zer_r, np_=np_, nc_=nc_)
    h1, tab1, dinv = _tc1(xp, deg[:np_, None], deg[np_:, None], w1p,
                          bn, np_)
    acc1 = _sc_aggregate(src1, dst1, tab1, zer_ch,
                         np_=np_, nc_=NC * nc_, d=DPAD1, colsplit=True)
    h3, hn3 = _tc2(acc1[:np_], acc1[np_:], h1, dinv, b1p, W2, bn, np_)
    acc2 = _sc_aggregate(src2, dst2, hn3, zer_cf,
                         np_=np_, nc_=nc_, d=DPAD2, colsplit=False)
    out = _tc3(acc2[:np_], acc2[np_:], h3, dinv, b2[None, :], Wl, bl[None, :],
               bn, np_)
    return out[:n]


# speedup vs baseline: 27.1345x; 2.0405x over previous
import functools

import jax
import jax.numpy as jnp
from jax import lax
from jax.experimental import pallas as pl
from jax.experimental.pallas import tpu as pltpu
from jax.experimental.pallas import tpu_sc as plsc

NC = 2
NS = 16
NW = NC * NS
CH = 128
DPAD1 = 16
DPAD2 = 8

F32 = jnp.float32


def _leaky(v):
    return jnp.where(v >= 0, v, 0.01 * v)


def _row_chunk(rpt):
    best = 8
    for czc in range(8, min(rpt, 512) + 1, 8):
        if rpt % czc == 0:
            best = czc
    return best


def _sc_mesh():
    return plsc.VectorSubcoreMesh(core_axis_name="c", subcore_axis_name="s",
                                  num_cores=NC, num_subcores=NS)


_SC_PARAMS = pltpu.CompilerParams(use_tc_tiling_on_sc=False)


U = 4


@functools.partial(jax.jit, static_argnames=("np_", "nc_"))
def _sc_degree(dst3, zeros_r, np_, nc_):
    rpt = np_ // NS

    def body(dst_hbm, zer_hbm, out_hbm, d0, d1, d2, d3, ones_v, buf_v,
             acc_sh, s0, s1, s2, s3):
        dsts = [d0, d1, d2, d3]
        sems = [s0, s1, s2, s3]
        c = lax.axis_index("c")
        s = lax.axis_index("s")
        w = c * NS + s
        base = pl.multiple_of(s * rpt, 8)
        pltpu.sync_copy(zer_hbm, buf_v)
        pltpu.sync_copy(buf_v, acc_sh.at[pl.ds(base, rpt)])
        for i in range(CH // 16):
            ones_v[pl.ds(16 * i, 16)] = jnp.full((16,), 1.0, F32)
        plsc.subcore_barrier()

        @pl.loop(0, nc_ // U)
        def _(t):
            hs = [pltpu.async_copy(dst_hbm.at[w, t * U + u], dsts[u], sems[u])
                  for u in range(U)]
            for u in range(U):
                hs[u].wait()
                pltpu.sync_copy(ones_v, acc_sh.at[dsts[u]], add=True)

        plsc.subcore_barrier()
        obase = pl.multiple_of(c * np_ + s * rpt, 8)
        pltpu.sync_copy(acc_sh.at[pl.ds(base, rpt)], buf_v)
        pltpu.sync_copy(buf_v, out_hbm.at[pl.ds(obase, rpt)])

    return pl.kernel(
        body,
        out_type=jax.ShapeDtypeStruct((NC * np_,), F32),
        mesh=_sc_mesh(),
        compiler_params=_SC_PARAMS,
        scratch_types=[
            pltpu.VMEM((CH,), jnp.int32),
            pltpu.VMEM((CH,), jnp.int32),
            pltpu.VMEM((CH,), jnp.int32),
            pltpu.VMEM((CH,), jnp.int32),
            pltpu.VMEM((CH,), F32),
            pltpu.VMEM((rpt,), F32),
            pltpu.VMEM_SHARED((np_,), F32),
            pltpu.SemaphoreType.DMA,
            pltpu.SemaphoreType.DMA,
            pltpu.SemaphoreType.DMA,
            pltpu.SemaphoreType.DMA,
        ],
    )(dst3, zeros_r)


@functools.partial(jax.jit, static_argnames=("np_", "nc_", "d", "colsplit"))
def _sc_aggregate(src3, dst3, table, zeros_cd, np_, nc_, d, colsplit):
    rpt = np_ // NS
    cz = _row_chunk(rpt)

    def body(src_hbm, dst_hbm, tab_hbm, zer_hbm, out_hbm,
             sv0, sv1, sv2, sv3, dv0, dv1, dv2, dv3, rv0, rv1, rv2, rv3,
             buf_v, acc_sh,
             is0, is1, is2, is3, js0, js1, js2, js3, gs0, gs1, gs2, gs3):
        srcs = [sv0, sv1, sv2, sv3]
        dsts = [dv0, dv1, dv2, dv3]
        rows = [rv0, rv1, rv2, rv3]
        isems = [is0, is1, is2, is3]
        jsems = [js0, js1, js2, js3]
        gsems = [gs0, gs1, gs2, gs3]
        c = lax.axis_index("c")
        s = lax.axis_index("s")
        w = c * NS + s
        pltpu.sync_copy(zer_hbm, buf_v)

        @pl.loop(0, rpt // cz)
        def _(k):
            zb = pl.multiple_of(s * rpt + k * cz, 8)
            pltpu.sync_copy(buf_v, acc_sh.at[pl.ds(zb, cz)])

        plsc.subcore_barrier()

        if colsplit:
            tab = tab_hbm.at[c]
            def chunk(hbm, j):
                return hbm.at[s, j]
        else:
            tab = tab_hbm
            def chunk(hbm, j):
                return hbm.at[w, j]

        @pl.loop(0, nc_ // U)
        def _(t):
            ihs = [(pltpu.async_copy(chunk(src_hbm, t * U + u), srcs[u],
                                     isems[u]),
                    pltpu.async_copy(chunk(dst_hbm, t * U + u), dsts[u],
                                     jsems[u]))
                   for u in range(U)]
            ghs = []
            for u in range(U):
                ihs[u][0].wait()
                ghs.append(pltpu.async_copy(tab.at[srcs[u]], rows[u],
                                            gsems[u]))
            for u in range(U):
                ghs[u].wait()
                ihs[u][1].wait()
                pltpu.sync_copy(rows[u], acc_sh.at[dsts[u]], add=True)

        plsc.subcore_barrier()

        @pl.loop(0, rpt // cz)
        def _(k):
            ib = pl.multiple_of(s * rpt + k * cz, 8)
            ob = pl.multiple_of(c * np_ + s * rpt + k * cz, 8)
            pltpu.sync_copy(acc_sh.at[pl.ds(ib, cz)], buf_v)
            pltpu.sync_copy(buf_v, out_hbm.at[pl.ds(ob, cz)])

    return pl.kernel(
        body,
        out_type=jax.ShapeDtypeStruct((NC * np_, d), F32),
        mesh=_sc_mesh(),
        compiler_params=_SC_PARAMS,
        scratch_types=(
            [pltpu.VMEM((CH,), jnp.int32)] * 8
            + [pltpu.VMEM((CH, d), F32)] * 4
            + [pltpu.VMEM((cz, d), F32),
               pltpu.VMEM_SHARED((np_, d), F32)]
            + [pltpu.SemaphoreType.DMA] * 12
        ),
    )(src3, dst3, table, zeros_cd)


def _tc1(xp, dega, degb, w1p, bn, np_):
    fin = xp.shape[1]
    fo = w1p.shape[1]
    half = fo // 2

    def body(x_ref, da_ref, db_ref, w_ref, h_ref, hn_ref, di_ref):
        deg = da_ref[...] + db_ref[...] + 1.0
        dinv = lax.rsqrt(deg)
        h = jnp.dot(x_ref[...], w_ref[...], preferred_element_type=F32)
        h_ref[...] = h
        hn = h * dinv
        zpad = jnp.zeros((h.shape[0], DPAD1 - half), F32)
        hn_ref[0] = jnp.concatenate([hn[:, :half], zpad], axis=1)
        hn_ref[1] = jnp.concatenate([hn[:, half:], zpad], axis=1)
        di_ref[...] = dinv

    return pl.pallas_call(
        body,
        grid=(np_ // bn,),
        in_specs=[
            pl.BlockSpec((bn, fin), lambda i: (i, 0)),
            pl.BlockSpec((bn, 1), lambda i: (i, 0)),
            pl.BlockSpec((bn, 1), lambda i: (i, 0)),
            pl.BlockSpec((fin, fo), lambda i: (0, 0)),
        ],
        out_specs=[
            pl.BlockSpec((bn, fo), lambda i: (i, 0)),
            pl.BlockSpec((NC, bn, DPAD1), lambda i: (0, i, 0)),
            pl.BlockSpec((bn, 1), lambda i: (i, 0)),
        ],
        out_shape=[
            jax.ShapeDtypeStruct((np_, fo), F32),
            jax.ShapeDtypeStruct((NC, np_, DPAD1), F32),
            jax.ShapeDtypeStruct((np_, 1), F32),
        ],
    )(xp, dega, degb, w1p)


def _tc2(acca, accb, h1, dinv, b1p, w2, bn, np_):
    fo = h1.shape[1]
    half = fo // 2
    f3 = w2.shape[1]

    def body(aa_ref, ab_ref, h_ref, di_ref, b_ref, w_ref, h3_ref, hn3_ref):
        dinv_c = di_ref[...]
        agg = jnp.concatenate([aa_ref[...][:, :half], ab_ref[...][:, :half]],
                              axis=1)
        out1 = dinv_c * agg + (dinv_c * dinv_c) * h_ref[...] + b_ref[...]
        out1 = _leaky(out1)
        h2 = jnp.maximum(out1[:, :half], out1[:, half:])
        h3 = jnp.dot(h2, w_ref[...], preferred_element_type=F32)
        h3_ref[...] = h3
        hn3 = h3 * dinv_c
        zpad = jnp.zeros((h3.shape[0], DPAD2 - f3), F32)
        hn3_ref[...] = jnp.concatenate([hn3, zpad], axis=1)

    return pl.pallas_call(
        body,
        grid=(np_ // bn,),
        in_specs=[
            pl.BlockSpec((bn, DPAD1), lambda i: (i, 0)),
            pl.BlockSpec((bn, DPAD1), lambda i: (i, 0)),
            pl.BlockSpec((bn, fo), lambda i: (i, 0)),
            pl.BlockSpec((bn, 1), lambda i: (i, 0)),
            pl.BlockSpec((1, fo), lambda i: (0, 0)),
            pl.BlockSpec((half, f3), lambda i: (0, 0)),
        ],
        out_specs=[
            pl.BlockSpec((bn, f3), lambda i: (i, 0)),
            pl.BlockSpec((bn, DPAD2), lambda i: (i, 0)),
        ],
        out_shape=[
            jax.ShapeDtypeStruct((np_, f3), F32),
            jax.ShapeDtypeStruct((np_, DPAD2), F32),
        ],
    )(acca, accb, h1, dinv, b1p, w2)


def _tc3(acca, accb, h3, dinv, b2, wl, bl, bn, np_):
    f3 = h3.shape[1]
    fl = wl.shape[1]

    def body(aa_ref, ab_ref, h_ref, di_ref, b2_ref, w_ref, bl_ref, o_ref):
        dinv_c = di_ref[...]
        agg = aa_ref[...][:, :f3] + ab_ref[...][:, :f3]
        out2 = dinv_c * agg + (dinv_c * dinv_c) * h_ref[...] + b2_ref[...]
        out2 = _leaky(out2)
        o_ref[...] = (jnp.dot(out2, w_ref[...], preferred_element_type=F32)
                      + bl_ref[...])

    return pl.pallas_call(
        body,
        grid=(np_ // bn,),
        in_specs=[
            pl.BlockSpec((bn, DPAD2), lambda i: (i, 0)),
            pl.BlockSpec((bn, DPAD2), lambda i: (i, 0)),
            pl.BlockSpec((bn, f3), lambda i: (i, 0)),
            pl.BlockSpec((bn, 1), lambda i: (i, 0)),
            pl.BlockSpec((1, f3), lambda i: (0, 0)),
            pl.BlockSpec((f3, fl), lambda i: (0, 0)),
            pl.BlockSpec((1, fl), lambda i: (0, 0)),
        ],
        out_specs=pl.BlockSpec((bn, fl), lambda i: (i, 0)),
        out_shape=jax.ShapeDtypeStruct((np_, fl), F32),
    )(acca, accb, h3, dinv, b2, wl, bl)


def kernel(x, edge_index, W1, b1, W2, b2, Wl, bl):
    n = x.shape[0]
    e = edge_index.shape[1]
    fo = W1.shape[1]
    half = fo // 2
    f3 = W2.shape[1]

    rpt = -(-(n + 1) // (NS * 8)) * 8
    np_ = rpt * NS
    nc_ = -(-(-(-e // (NW * CH))) // U) * U
    ep = NW * CH * nc_

    perm = jnp.arange(fo).reshape(fo // 2, 2).T.reshape(fo)
    w1p = W1[:, perm]
    b1p = b1[perm][None, :]

    src_f = jnp.concatenate(
        [edge_index[0], jnp.full((ep - e,), n, jnp.int32)])
    dst_f = jnp.concatenate(
        [edge_index[1], jnp.full((ep - e,), n, jnp.int32)])
    src2 = src_f.reshape(NW, nc_, CH)
    dst2 = dst_f.reshape(NW, nc_, CH)
    src1 = src_f.reshape(NS, NC * nc_, CH)
    dst1 = dst_f.reshape(NS, NC * nc_, CH)
    xp = jnp.concatenate(
        [x, jnp.zeros((np_ - n, x.shape[1]), F32)], axis=0)

    cz = _row_chunk(rpt)
    zer_r = jnp.zeros((rpt,), F32)
    zer_ch = jnp.zeros((cz, DPAD1), F32)
    zer_cf = jnp.zeros((cz, DPAD2), F32)

    bn = np_ // 16

    deg = _sc_degree(dst2, zer_r, np_=np_, nc_=nc_)
    h1, tab1, dinv = _tc1(xp, deg[:np_, None], deg[np_:, None], w1p,
                          bn, np_)
    acc1 = _sc_aggregate(src1, dst1, tab1, zer_ch,
                         np_=np_, nc_=NC * nc_, d=DPAD1, colsplit=True)
    h3, hn3 = _tc2(acc1[:np_], acc1[np_:], h1, dinv, b1p, W2, bn, np_)
    acc2 = _sc_aggregate(src2, dst2, hn3, zer_cf,
                         np_=np_, nc_=nc_, d=DPAD2, colsplit=False)
    out = _tc3(acc2[:np_], acc2[np_:], h3, dinv, b2[None, :], Wl, bl[None, :],
               bn, np_)
    return out[:n]


# speedup vs baseline: 30.3888x; 1.1199x over previous
import functools

import jax
import jax.numpy as jnp
from jax import lax
from jax.experimental import pallas as pl
from jax.experimental.pallas import tpu as pltpu
from jax.experimental.pallas import tpu_sc as plsc

NC = 2
NS = 16
NW = NC * NS
CH = 128
DPAD1 = 16
DPAD2 = 8

F32 = jnp.float32


def _leaky(v):
    return jnp.where(v >= 0, v, 0.01 * v)


def _row_chunk(rpt):
    best = 8
    for czc in range(8, min(rpt, 512) + 1, 8):
        if rpt % czc == 0:
            best = czc
    return best


def _sc_mesh():
    return plsc.VectorSubcoreMesh(core_axis_name="c", subcore_axis_name="s",
                                  num_cores=NC, num_subcores=NS)


_SC_PARAMS = pltpu.CompilerParams(use_tc_tiling_on_sc=False)


U = 4


@functools.partial(jax.jit, static_argnames=("np_", "nc_"))
def _sc_degree(dst3, zeros_r, np_, nc_):
    rpt = np_ // NS

    def body(dst_hbm, zer_hbm, out_hbm, d0, d1, d2, d3, ones_v, buf_v,
             acc_sh, s0, s1, s2, s3):
        dsts = [d0, d1, d2, d3]
        sems = [s0, s1, s2, s3]
        c = lax.axis_index("c")
        s = lax.axis_index("s")
        w = c * NS + s
        base = pl.multiple_of(s * rpt, 8)
        pltpu.sync_copy(zer_hbm, buf_v)
        pltpu.sync_copy(buf_v, acc_sh.at[pl.ds(base, rpt)])
        for i in range(CH // 16):
            ones_v[pl.ds(16 * i, 16)] = jnp.full((16,), 1.0, F32)
        plsc.subcore_barrier()

        @pl.loop(0, nc_ // U)
        def _(t):
            hs = [pltpu.async_copy(dst_hbm.at[w, t * U + u], dsts[u], sems[u])
                  for u in range(U)]
            for u in range(U):
                hs[u].wait()
                pltpu.sync_copy(ones_v, acc_sh.at[dsts[u]], add=True)

        plsc.subcore_barrier()
        obase = pl.multiple_of(c * np_ + s * rpt, 8)
        pltpu.sync_copy(acc_sh.at[pl.ds(base, rpt)], buf_v)
        pltpu.sync_copy(buf_v, out_hbm.at[pl.ds(obase, rpt)])

    return pl.kernel(
        body,
        out_type=jax.ShapeDtypeStruct((NC * np_,), F32),
        mesh=_sc_mesh(),
        compiler_params=_SC_PARAMS,
        scratch_types=[
            pltpu.VMEM((CH,), jnp.int32),
            pltpu.VMEM((CH,), jnp.int32),
            pltpu.VMEM((CH,), jnp.int32),
            pltpu.VMEM((CH,), jnp.int32),
            pltpu.VMEM((CH,), F32),
            pltpu.VMEM((rpt,), F32),
            pltpu.VMEM_SHARED((np_,), F32),
            pltpu.SemaphoreType.DMA,
            pltpu.SemaphoreType.DMA,
            pltpu.SemaphoreType.DMA,
            pltpu.SemaphoreType.DMA,
        ],
    )(dst3, zeros_r)


@functools.partial(jax.jit, static_argnames=("np_", "nc_", "d", "colsplit"))
def _sc_aggregate(src3, dst3, table, zeros_cd, np_, nc_, d, colsplit):
    rpt = np_ // NS
    cz = _row_chunk(rpt)

    def body(src_hbm, dst_hbm, tab_hbm, zer_hbm, out_hbm,
             sv0, sv1, sv2, sv3, dv0, dv1, dv2, dv3, rv0, rv1, rv2, rv3,
             buf_v, acc_sh,
             is0, is1, is2, is3, js0, js1, js2, js3, gs0, gs1, gs2, gs3):
        srcs = [sv0, sv1, sv2, sv3]
        dsts = [dv0, dv1, dv2, dv3]
        rows = [rv0, rv1, rv2, rv3]
        isems = [is0, is1, is2, is3]
        jsems = [js0, js1, js2, js3]
        gsems = [gs0, gs1, gs2, gs3]
        c = lax.axis_index("c")
        s = lax.axis_index("s")
        w = c * NS + s
        pltpu.sync_copy(zer_hbm, buf_v)

        @pl.loop(0, rpt // cz)
        def _(k):
            zb = pl.multiple_of(s * rpt + k * cz, 8)
            pltpu.sync_copy(buf_v, acc_sh.at[pl.ds(zb, cz)])

        plsc.subcore_barrier()

        if colsplit:
            tab = tab_hbm.at[c]
            def chunk(hbm, j):
                return hbm.at[s, j]
        else:
            tab = tab_hbm
            def chunk(hbm, j):
                return hbm.at[w, j]

        @pl.loop(0, nc_ // U)
        def _(t):
            ihs = [(pltpu.async_copy(chunk(src_hbm, t * U + u), srcs[u],
                                     isems[u]),
                    pltpu.async_copy(chunk(dst_hbm, t * U + u), dsts[u],
                                     jsems[u]))
                   for u in range(U)]
            ghs = []
            for u in range(U):
                ihs[u][0].wait()
                ghs.append(pltpu.async_copy(tab.at[srcs[u]], rows[u],
                                            gsems[u]))
            for u in range(U):
                ghs[u].wait()
                ihs[u][1].wait()
                pltpu.sync_copy(rows[u], acc_sh.at[dsts[u]], add=True)

        plsc.subcore_barrier()

        @pl.loop(0, rpt // cz)
        def _(k):
            ib = pl.multiple_of(s * rpt + k * cz, 8)
            ob = pl.multiple_of(c * np_ + s * rpt + k * cz, 8)
            pltpu.sync_copy(acc_sh.at[pl.ds(ib, cz)], buf_v)
            pltpu.sync_copy(buf_v, out_hbm.at[pl.ds(ob, cz)])

    return pl.kernel(
        body,
        out_type=jax.ShapeDtypeStruct((NC * np_, d), F32),
        mesh=_sc_mesh(),
        compiler_params=_SC_PARAMS,
        scratch_types=(
            [pltpu.VMEM((CH,), jnp.int32)] * 8
            + [pltpu.VMEM((CH, d), F32)] * 4
            + [pltpu.VMEM((cz, d), F32),
               pltpu.VMEM_SHARED((np_, d), F32)]
            + [pltpu.SemaphoreType.DMA] * 12
        ),
    )(src3, dst3, table, zeros_cd)


def _tc1(xp, deg2, w1p, bn, np_):
    fin = xp.shape[1]
    fo = w1p.shape[1]
    half = fo // 2
    nb = np_ // bn

    def body(x_ref, da_ref, db_ref, w_ref, h_ref, hn_ref, di_ref):
        deg = da_ref[...] + db_ref[...] + 1.0
        dinv = lax.rsqrt(deg)
        h = jnp.dot(x_ref[...], w_ref[...], preferred_element_type=F32)
        h_ref[...] = h
        hn = h * dinv
        zpad = jnp.zeros((h.shape[0], DPAD1 - half), F32)
        hn_ref[0] = jnp.concatenate([hn[:, :half], zpad], axis=1)
        hn_ref[1] = jnp.concatenate([hn[:, half:], zpad], axis=1)
        di_ref[...] = dinv

    return pl.pallas_call(
        body,
        grid=(np_ // bn,),
        in_specs=[
            pl.BlockSpec((bn, fin), lambda i: (i, 0)),
            pl.BlockSpec((bn, 1), lambda i: (i, 0)),
            pl.BlockSpec((bn, 1), lambda i, nb=nb: (i + nb, 0)),
            pl.BlockSpec((fin, fo), lambda i: (0, 0)),
        ],
        out_specs=[
            pl.BlockSpec((bn, fo), lambda i: (i, 0)),
            pl.BlockSpec((NC, bn, DPAD1), lambda i: (0, i, 0)),
            pl.BlockSpec((bn, 1), lambda i: (i, 0)),
        ],
        out_shape=[
            jax.ShapeDtypeStruct((np_, fo), F32),
            jax.ShapeDtypeStruct((NC, np_, DPAD1), F32),
            jax.ShapeDtypeStruct((np_, 1), F32),
        ],
    )(xp, deg2, deg2, w1p)


def _tc2(acc, h1, dinv, b1p, w2, bn, np_):
    fo = h1.shape[1]
    half = fo // 2
    f3 = w2.shape[1]
    nb = np_ // bn

    def body(aa_ref, ab_ref, h_ref, di_ref, b_ref, w_ref, h3_ref, hn3_ref):
        dinv_c = di_ref[...]
        agg = jnp.concatenate([aa_ref[...][:, :half], ab_ref[...][:, :half]],
                              axis=1)
        out1 = dinv_c * agg + (dinv_c * dinv_c) * h_ref[...] + b_ref[...]
        out1 = _leaky(out1)
        h2 = jnp.maximum(out1[:, :half], out1[:, half:])
        h3 = jnp.dot(h2, w_ref[...], preferred_element_type=F32)
        h3_ref[...] = h3
        hn3 = h3 * dinv_c
        zpad = jnp.zeros((h3.shape[0], DPAD2 - f3), F32)
        hn3_ref[...] = jnp.concatenate([hn3, zpad], axis=1)

    return pl.pallas_call(
        body,
        grid=(np_ // bn,),
        in_specs=[
            pl.BlockSpec((bn, DPAD1), lambda i: (i, 0)),
            pl.BlockSpec((bn, DPAD1), lambda i, nb=nb: (i + nb, 0)),
            pl.BlockSpec((bn, fo), lambda i: (i, 0)),
            pl.BlockSpec((bn, 1), lambda i: (i, 0)),
            pl.BlockSpec((1, fo), lambda i: (0, 0)),
            pl.BlockSpec((half, f3), lambda i: (0, 0)),
        ],
        out_specs=[
            pl.BlockSpec((bn, f3), lambda i: (i, 0)),
            pl.BlockSpec((bn, DPAD2), lambda i: (i, 0)),
        ],
        out_shape=[
            jax.ShapeDtypeStruct((np_, f3), F32),
            jax.ShapeDtypeStruct((np_, DPAD2), F32),
        ],
    )(acc, acc, h1, dinv, b1p, w2)


def _tc3(acc, h3, dinv, b2, wl, bl, bn, np_, n):
    f3 = h3.shape[1]
    fl = wl.shape[1]
    nb = np_ // bn

    def body(aa_ref, ab_ref, h_ref, di_ref, b2_ref, w_ref, bl_ref, o_ref):
        dinv_c = di_ref[...]
        agg = aa_ref[...][:, :f3] + ab_ref[...][:, :f3]
        out2 = dinv_c * agg + (dinv_c * dinv_c) * h_ref[...] + b2_ref[...]
        out2 = _leaky(out2)
        o_ref[...] = (jnp.dot(out2, w_ref[...], preferred_element_type=F32)
                      + bl_ref[...])

    return pl.pallas_call(
        body,
        grid=(np_ // bn,),
        in_specs=[
            pl.BlockSpec((bn, DPAD2), lambda i: (i, 0)),
            pl.BlockSpec((bn, DPAD2), lambda i, nb=nb: (i + nb, 0)),
            pl.BlockSpec((bn, f3), lambda i: (i, 0)),
            pl.BlockSpec((bn, 1), lambda i: (i, 0)),
            pl.BlockSpec((1, f3), lambda i: (0, 0)),
            pl.BlockSpec((f3, fl), lambda i: (0, 0)),
            pl.BlockSpec((1, fl), lambda i: (0, 0)),
        ],
        out_specs=pl.BlockSpec((bn, fl), lambda i: (i, 0)),
        out_shape=jax.ShapeDtypeStruct((n, fl), F32),
    )(acc, acc, h3, dinv, b2, wl, bl)


def kernel(x, edge_index, W1, b1, W2, b2, Wl, bl):
    n = x.shape[0]
    e = edge_index.shape[1]
    fo = W1.shape[1]
    half = fo // 2
    f3 = W2.shape[1]

    rpt = -(-(n + 1) // (NS * 8)) * 8
    np_ = rpt * NS
    nc_ = -(-(-(-e // (NW * CH))) // U) * U
    ep = NW * CH * nc_

    perm = jnp.arange(fo).reshape(fo // 2, 2).T.reshape(fo)
    w1p = W1[:, perm]
    b1p = b1[perm][None, :]

    src_f = jnp.concatenate(
        [edge_index[0], jnp.full((ep - e,), n, jnp.int32)])
    dst_f = jnp.concatenate(
        [edge_index[1], jnp.full((ep - e,), n, jnp.int32)])
    src2 = src_f.reshape(NW, nc_, CH)
    dst2 = dst_f.reshape(NW, nc_, CH)
    src1 = src_f.reshape(NS, NC * nc_, CH)
    dst1 = dst_f.reshape(NS, NC * nc_, CH)
    xp = jnp.concatenate(
        [x, jnp.zeros((np_ - n, x.shape[1]), F32)], axis=0)

    cz = _row_chunk(rpt)
    zer_r = jnp.zeros((rpt,), F32)
    zer_ch = jnp.zeros((cz, DPAD1), F32)
    zer_cf = jnp.zeros((cz, DPAD2), F32)

    bn = np_ // 16

    deg = _sc_degree(dst2, zer_r, np_=np_, nc_=nc_)
    h1, tab1, dinv = _tc1(xp, deg.reshape(NC * np_, 1), w1p, bn, np_)
    acc1 = _sc_aggregate(src1, dst1, tab1, zer_ch,
                         np_=np_, nc_=NC * nc_, d=DPAD1, colsplit=True)
    h3, hn3 = _tc2(acc1, h1, dinv, b1p, W2, bn, np_)
    acc2 = _sc_aggregate(src2, dst2, hn3, zer_cf,
                         np_=np_, nc_=nc_, d=DPAD2, colsplit=False)
    return _tc3(acc2, h3, dinv, b2[None, :], Wl, bl[None, :], bn, np_, n)


# speedup vs baseline: 30.4659x; 1.0025x over previous
import functools

import jax
import jax.numpy as jnp
from jax import lax
from jax.experimental import pallas as pl
from jax.experimental.pallas import tpu as pltpu
from jax.experimental.pallas import tpu_sc as plsc

NC = 2
NS = 16
NW = NC * NS
CH = 128
DPAD1 = 16
DPAD2 = 8

F32 = jnp.float32


def _leaky(v):
    return jnp.where(v >= 0, v, 0.01 * v)


def _row_chunk(rpt):
    best = 8
    for czc in range(8, min(rpt, 512) + 1, 8):
        if rpt % czc == 0:
            best = czc
    return best


def _sc_mesh():
    return plsc.VectorSubcoreMesh(core_axis_name="c", subcore_axis_name="s",
                                  num_cores=NC, num_subcores=NS)


_SC_PARAMS = pltpu.CompilerParams(use_tc_tiling_on_sc=False)


U = 4


@functools.partial(jax.jit, static_argnames=("np_", "nc_"))
def _sc_degree(dst3, zeros_r, np_, nc_):
    rpt = np_ // NS

    def body(dst_hbm, zer_hbm, out_hbm, d0, d1, d2, d3, ones_v, buf_v,
             acc_sh, s0, s1, s2, s3):
        dsts = [d0, d1, d2, d3]
        sems = [s0, s1, s2, s3]
        c = lax.axis_index("c")
        s = lax.axis_index("s")
        w = c * NS + s
        base = pl.multiple_of(s * rpt, 8)
        pltpu.sync_copy(zer_hbm, buf_v)
        pltpu.sync_copy(buf_v, acc_sh.at[pl.ds(base, rpt)])
        for i in range(CH // 16):
            ones_v[pl.ds(16 * i, 16)] = jnp.full((16,), 1.0, F32)
        plsc.subcore_barrier()

        @pl.loop(0, nc_ // U)
        def _(t):
            hs = [pltpu.async_copy(dst_hbm.at[w, t * U + u], dsts[u], sems[u])
                  for u in range(U)]
            for u in range(U):
                hs[u].wait()
                pltpu.sync_copy(ones_v, acc_sh.at[dsts[u]], add=True)

        plsc.subcore_barrier()
        obase = pl.multiple_of(c * np_ + s * rpt, 8)
        pltpu.sync_copy(acc_sh.at[pl.ds(base, rpt)], buf_v)
        pltpu.sync_copy(buf_v, out_hbm.at[pl.ds(obase, rpt)])

    return pl.kernel(
        body,
        out_type=jax.ShapeDtypeStruct((NC * np_,), F32),
        mesh=_sc_mesh(),
        compiler_params=_SC_PARAMS,
        scratch_types=[
            pltpu.VMEM((CH,), jnp.int32),
            pltpu.VMEM((CH,), jnp.int32),
            pltpu.VMEM((CH,), jnp.int32),
            pltpu.VMEM((CH,), jnp.int32),
            pltpu.VMEM((CH,), F32),
            pltpu.VMEM((rpt,), F32),
            pltpu.VMEM_SHARED((np_,), F32),
            pltpu.SemaphoreType.DMA,
            pltpu.SemaphoreType.DMA,
            pltpu.SemaphoreType.DMA,
            pltpu.SemaphoreType.DMA,
        ],
    )(dst3, zeros_r)


@functools.partial(jax.jit, static_argnames=("np_", "nc_", "d", "colsplit"))
def _sc_aggregate(src3, dst3, table, zeros_cd, np_, nc_, d, colsplit):
    rpt = np_ // NS
    cz = _row_chunk(rpt)

    def body(src_hbm, dst_hbm, tab_hbm, zer_hbm, out_hbm,
             sv0, sv1, sv2, sv3, dv0, dv1, dv2, dv3, rv0, rv1, rv2, rv3,
             buf_v, acc_sh,
             is0, is1, is2, is3, js0, js1, js2, js3, gs0, gs1, gs2, gs3):
        srcs = [sv0, sv1, sv2, sv3]
        dsts = [dv0, dv1, dv2, dv3]
        rows = [rv0, rv1, rv2, rv3]
        isems = [is0, is1, is2, is3]
        jsems = [js0, js1, js2, js3]
        gsems = [gs0, gs1, gs2, gs3]
        c = lax.axis_index("c")
        s = lax.axis_index("s")
        w = c * NS + s
        pltpu.sync_copy(zer_hbm, buf_v)

        @pl.loop(0, rpt // cz)
        def _(k):
            zb = pl.multiple_of(s * rpt + k * cz, 8)
            pltpu.sync_copy(buf_v, acc_sh.at[pl.ds(zb, cz)])

        plsc.subcore_barrier()

        if colsplit:
            tab = tab_hbm.at[c]
            def chunk(hbm, j):
                return hbm.at[s, j]
        else:
            tab = tab_hbm
            def chunk(hbm, j):
                return hbm.at[w, j]

        @pl.loop(0, nc_ // U)
        def _(t):
            ihs = [(pltpu.async_copy(chunk(src_hbm, t * U + u), srcs[u],
                                     isems[u]),
                    pltpu.async_copy(chunk(dst_hbm, t * U + u), dsts[u],
                                     jsems[u]))
                   for u in range(U)]
            ghs = []
            for u in range(U):
                ihs[u][0].wait()
                ghs.append(pltpu.async_copy(tab.at[srcs[u]], rows[u],
                                            gsems[u]))
            for u in range(U):
                ghs[u].wait()
                ihs[u][1].wait()
                pltpu.sync_copy(rows[u], acc_sh.at[dsts[u]], add=True)

        plsc.subcore_barrier()

        @pl.loop(0, rpt // cz)
        def _(k):
            ib = pl.multiple_of(s * rpt + k * cz, 8)
            ob = pl.multiple_of(c * np_ + s * rpt + k * cz, 8)
            pltpu.sync_copy(acc_sh.at[pl.ds(ib, cz)], buf_v)
            pltpu.sync_copy(buf_v, out_hbm.at[pl.ds(ob, cz)])

    return pl.kernel(
        body,
        out_type=jax.ShapeDtypeStruct((NC * np_, d), F32),
        mesh=_sc_mesh(),
        compiler_params=_SC_PARAMS,
        scratch_types=(
            [pltpu.VMEM((CH,), jnp.int32)] * 8
            + [pltpu.VMEM((CH, d), F32)] * 4
            + [pltpu.VMEM((cz, d), F32),
               pltpu.VMEM_SHARED((np_, d), F32)]
            + [pltpu.SemaphoreType.DMA] * 12
        ),
    )(src3, dst3, table, zeros_cd)


def _tc0(xp, w1p, bn, np_):
    fin = xp.shape[1]
    fo = w1p.shape[1]

    def body(x_ref, w_ref, h_ref):
        h_ref[...] = jnp.dot(x_ref[...], w_ref[...],
                             preferred_element_type=F32)

    return pl.pallas_call(
        body,
        grid=(np_ // bn,),
        in_specs=[
            pl.BlockSpec((bn, fin), lambda i: (i, 0)),
            pl.BlockSpec((fin, fo), lambda i: (0, 0)),
        ],
        out_specs=pl.BlockSpec((bn, fo), lambda i: (i, 0)),
        out_shape=jax.ShapeDtypeStruct((np_, fo), F32),
    )(xp, w1p)


def _tc1(h1, deg2, bn, np_):
    fo = h1.shape[1]
    half = fo // 2
    nb = np_ // bn

    def body(h_ref, da_ref, db_ref, hn_ref, di_ref):
        deg = da_ref[...] + db_ref[...] + 1.0
        dinv = lax.rsqrt(deg)
        h = h_ref[...]
        hn = h * dinv
        zpad = jnp.zeros((h.shape[0], DPAD1 - half), F32)
        hn_ref[0] = jnp.concatenate([hn[:, :half], zpad], axis=1)
        hn_ref[1] = jnp.concatenate([hn[:, half:], zpad], axis=1)
        di_ref[...] = dinv

    return pl.pallas_call(
        body,
        grid=(np_ // bn,),
        in_specs=[
            pl.BlockSpec((bn, fo), lambda i: (i, 0)),
            pl.BlockSpec((bn, 1), lambda i: (i, 0)),
            pl.BlockSpec((bn, 1), lambda i, nb=nb: (i + nb, 0)),
        ],
        out_specs=[
            pl.BlockSpec((NC, bn, DPAD1), lambda i: (0, i, 0)),
            pl.BlockSpec((bn, 1), lambda i: (i, 0)),
        ],
        out_shape=[
            jax.ShapeDtypeStruct((NC, np_, DPAD1), F32),
            jax.ShapeDtypeStruct((np_, 1), F32),
        ],
    )(h1, deg2, deg2)


def _tc2(acc, h1, dinv, b1p, w2, bn, np_):
    fo = h1.shape[1]
    half = fo // 2
    f3 = w2.shape[1]
    nb = np_ // bn

    def body(aa_ref, ab_ref, h_ref, di_ref, b_ref, w_ref, h3_ref, hn3_ref):
        dinv_c = di_ref[...]
        agg = jnp.concatenate([aa_ref[...][:, :half], ab_ref[...][:, :half]],
                              axis=1)
        out1 = dinv_c * agg + (dinv_c * dinv_c) * h_ref[...] + b_ref[...]
        out1 = _leaky(out1)
        h2 = jnp.maximum(out1[:, :half], out1[:, half:])
        h3 = jnp.dot(h2, w_ref[...], preferred_element_type=F32)
        h3_ref[...] = h3
        hn3 = h3 * dinv_c
        zpad = jnp.zeros((h3.shape[0], DPAD2 - f3), F32)
        hn3_ref[...] = jnp.concatenate([hn3, zpad], axis=1)

    return pl.pallas_call(
        body,
        grid=(np_ // bn,),
        in_specs=[
            pl.BlockSpec((bn, DPAD1), lambda i: (i, 0)),
            pl.BlockSpec((bn, DPAD1), lambda i, nb=nb: (i + nb, 0)),
            pl.BlockSpec((bn, fo), lambda i: (i, 0)),
            pl.BlockSpec((bn, 1), lambda i: (i, 0)),
            pl.BlockSpec((1, fo), lambda i: (0, 0)),
            pl.BlockSpec((half, f3), lambda i: (0, 0)),
        ],
        out_specs=[
            pl.BlockSpec((bn, f3), lambda i: (i, 0)),
            pl.BlockSpec((bn, DPAD2), lambda i: (i, 0)),
        ],
        out_shape=[
            jax.ShapeDtypeStruct((np_, f3), F32),
            jax.ShapeDtypeStruct((np_, DPAD2), F32),
        ],
    )(acc, acc, h1, dinv, b1p, w2)


def _tc3(acc, h3, dinv, b2, wl, bl, bn, np_, n):
    f3 = h3.shape[1]
    fl = wl.shape[1]
    nb = np_ // bn

    def body(aa_ref, ab_ref, h_ref, di_ref, b2_ref, w_ref, bl_ref, o_ref):
        dinv_c = di_ref[...]
        agg = aa_ref[...][:, :f3] + ab_ref[...][:, :f3]
        out2 = dinv_c * agg + (dinv_c * dinv_c) * h_ref[...] + b2_ref[...]
        out2 = _leaky(out2)
        o_ref[...] = (jnp.dot(out2, w_ref[...], preferred_element_type=F32)
                      + bl_ref[...])

    return pl.pallas_call(
        body,
        grid=(np_ // bn,),
        in_specs=[
            pl.BlockSpec((bn, DPAD2), lambda i: (i, 0)),
            pl.BlockSpec((bn, DPAD2), lambda i, nb=nb: (i + nb, 0)),
            pl.BlockSpec((bn, f3), lambda i: (i, 0)),
            pl.BlockSpec((bn, 1), lambda i: (i, 0)),
            pl.BlockSpec((1, f3), lambda i: (0, 0)),
            pl.BlockSpec((f3, fl), lambda i: (0, 0)),
            pl.BlockSpec((1, fl), lambda i: (0, 0)),
        ],
        out_specs=pl.BlockSpec((bn, fl), lambda i: (i, 0)),
        out_shape=jax.ShapeDtypeStruct((n, fl), F32),
    )(acc, acc, h3, dinv, b2, wl, bl)


def kernel(x, edge_index, W1, b1, W2, b2, Wl, bl):
    n = x.shape[0]
    e = edge_index.shape[1]
    fo = W1.shape[1]
    half = fo // 2
    f3 = W2.shape[1]

    rpt = -(-(n + 1) // (NS * 8)) * 8
    np_ = rpt * NS
    nc_ = -(-(-(-e // (NW * CH))) // U) * U
    ep = NW * CH * nc_

    perm = jnp.arange(fo).reshape(fo // 2, 2).T.reshape(fo)
    w1p = W1[:, perm]
    b1p = b1[perm][None, :]

    src_f = jnp.concatenate(
        [edge_index[0], jnp.full((ep - e,), n, jnp.int32)])
    dst_f = jnp.concatenate(
        [edge_index[1], jnp.full((ep - e,), n, jnp.int32)])
    src2 = src_f.reshape(NW, nc_, CH)
    dst2 = dst_f.reshape(NW, nc_, CH)
    src1 = src_f.reshape(NS, NC * nc_, CH)
    dst1 = dst_f.reshape(NS, NC * nc_, CH)
    xp = jnp.concatenate(
        [x, jnp.zeros((np_ - n, x.shape[1]), F32)], axis=0)

    cz = _row_chunk(rpt)
    zer_r = jnp.zeros((rpt,), F32)
    zer_ch = jnp.zeros((cz, DPAD1), F32)
    zer_cf = jnp.zeros((cz, DPAD2), F32)

    bn = np_ // 16

    h1 = _tc0(xp, w1p, bn, np_)
    deg = _sc_degree(dst2, zer_r, np_=np_, nc_=nc_)
    tab1, dinv = _tc1(h1, deg.reshape(NC * np_, 1), bn, np_)
    acc1 = _sc_aggregate(src1, dst1, tab1, zer_ch,
                         np_=np_, nc_=NC * nc_, d=DPAD1, colsplit=True)
    h3, hn3 = _tc2(acc1, h1, dinv, b1p, W2, bn, np_)
    acc2 = _sc_aggregate(src2, dst2, hn3, zer_cf,
                         np_=np_, nc_=nc_, d=DPAD2, colsplit=False)
    return _tc3(acc2, h3, dinv, b2[None, :], Wl, bl[None, :], bn, np_, n)


# speedup vs baseline: 30.4723x; 1.0002x over previous
import functools

import jax
import jax.numpy as jnp
from jax import lax
from jax.experimental import pallas as pl
from jax.experimental.pallas import tpu as pltpu
from jax.experimental.pallas import tpu_sc as plsc

NC = 2
NS = 16
NW = NC * NS
CH = 128
DPAD1 = 16
DPAD2 = 8

F32 = jnp.float32


def _leaky(v):
    return jnp.where(v >= 0, v, 0.01 * v)


def _row_chunk(rpt):
    best = 8
    for czc in range(8, min(rpt, 512) + 1, 8):
        if rpt % czc == 0:
            best = czc
    return best


def _sc_mesh():
    return plsc.VectorSubcoreMesh(core_axis_name="c", subcore_axis_name="s",
                                  num_cores=NC, num_subcores=NS)


_SC_PARAMS = pltpu.CompilerParams(use_tc_tiling_on_sc=False)


U = 4


@functools.partial(jax.jit, static_argnames=("np_", "nc_"))
def _sc_degree(dst3, zeros_r, np_, nc_):
    rpt = np_ // NS

    def body(dst_hbm, zer_hbm, out_hbm, d0, d1, d2, d3, ones_v, buf_v,
             acc_sh, s0, s1, s2, s3):
        dsts = [d0, d1, d2, d3]
        sems = [s0, s1, s2, s3]
        c = lax.axis_index("c")
        s = lax.axis_index("s")
        w = c * NS + s
        base = pl.multiple_of(s * rpt, 8)
        pltpu.sync_copy(zer_hbm, buf_v)
        pltpu.sync_copy(buf_v, acc_sh.at[pl.ds(base, rpt)])
        for i in range(CH // 16):
            ones_v[pl.ds(16 * i, 16)] = jnp.full((16,), 1.0, F32)
        plsc.subcore_barrier()

        @pl.loop(0, nc_ // U)
        def _(t):
            hs = [pltpu.async_copy(dst_hbm.at[w, t * U + u], dsts[u], sems[u])
                  for u in range(U)]
            for u in range(U):
                hs[u].wait()
                pltpu.sync_copy(ones_v, acc_sh.at[dsts[u]], add=True)

        plsc.subcore_barrier()
        obase = pl.multiple_of(c * np_ + s * rpt, 8)
        pltpu.sync_copy(acc_sh.at[pl.ds(base, rpt)], buf_v)
        pltpu.sync_copy(buf_v, out_hbm.at[pl.ds(obase, rpt)])

    return pl.kernel(
        body,
        out_type=jax.ShapeDtypeStruct((NC * np_,), F32),
        mesh=_sc_mesh(),
        compiler_params=_SC_PARAMS,
        scratch_types=[
            pltpu.VMEM((CH,), jnp.int32),
            pltpu.VMEM((CH,), jnp.int32),
            pltpu.VMEM((CH,), jnp.int32),
            pltpu.VMEM((CH,), jnp.int32),
            pltpu.VMEM((CH,), F32),
            pltpu.VMEM((rpt,), F32),
            pltpu.VMEM_SHARED((np_,), F32),
            pltpu.SemaphoreType.DMA,
            pltpu.SemaphoreType.DMA,
            pltpu.SemaphoreType.DMA,
            pltpu.SemaphoreType.DMA,
        ],
    )(dst3, zeros_r)


@functools.partial(jax.jit, static_argnames=("np_", "nc_", "d", "colsplit"))
def _sc_aggregate(src3, dst3, table, zeros_cd, np_, nc_, d, colsplit):
    rpt = np_ // NS
    cz = _row_chunk(rpt)

    def body(src_hbm, dst_hbm, tab_hbm, zer_hbm, out_hbm,
             sv0, sv1, sv2, sv3, dv0, dv1, dv2, dv3, rv0, rv1, rv2, rv3,
             buf_v, acc_sh,
             is0, is1, is2, is3, js0, js1, js2, js3, gs0, gs1, gs2, gs3):
        srcs = [sv0, sv1, sv2, sv3]
        dsts = [dv0, dv1, dv2, dv3]
        rows = [rv0, rv1, rv2, rv3]
        isems = [is0, is1, is2, is3]
        jsems = [js0, js1, js2, js3]
        gsems = [gs0, gs1, gs2, gs3]
        c = lax.axis_index("c")
        s = lax.axis_index("s")
        w = c * NS + s
        pltpu.sync_copy(zer_hbm, buf_v)

        @pl.loop(0, rpt // cz)
        def _(k):
            zb = pl.multiple_of(s * rpt + k * cz, 8)
            pltpu.sync_copy(buf_v, acc_sh.at[pl.ds(zb, cz)])

        plsc.subcore_barrier()

        if colsplit:
            tab = tab_hbm.at[c]
            def chunk(hbm, j):
                return hbm.at[s, j]
        else:
            tab = tab_hbm
            def chunk(hbm, j):
                return hbm.at[w, j]

        @pl.loop(0, nc_ // U)
        def _(t):
            ihs = [(pltpu.async_copy(chunk(src_hbm, t * U + u), srcs[u],
                                     isems[u]),
                    pltpu.async_copy(chunk(dst_hbm, t * U + u), dsts[u],
                                     jsems[u]))
                   for u in range(U)]
            ghs = []
            for u in range(U):
                ihs[u][0].wait()
                ghs.append(pltpu.async_copy(tab.at[srcs[u]], rows[u],
                                            gsems[u]))
            for u in range(U):
                ghs[u].wait()
                ihs[u][1].wait()
                pltpu.sync_copy(rows[u], acc_sh.at[dsts[u]], add=True)

        plsc.subcore_barrier()

        @pl.loop(0, rpt // cz)
        def _(k):
            ib = pl.multiple_of(s * rpt + k * cz, 8)
            ob = pl.multiple_of(c * np_ + s * rpt + k * cz, 8)
            pltpu.sync_copy(acc_sh.at[pl.ds(ib, cz)], buf_v)
            pltpu.sync_copy(buf_v, out_hbm.at[pl.ds(ob, cz)])

    return pl.kernel(
        body,
        out_type=jax.ShapeDtypeStruct((NC * np_, d), F32),
        mesh=_sc_mesh(),
        compiler_params=_SC_PARAMS,
        scratch_types=(
            [pltpu.VMEM((CH,), jnp.int32)] * 8
            + [pltpu.VMEM((CH, d), F32)] * 4
            + [pltpu.VMEM((cz, d), F32),
               pltpu.VMEM_SHARED((np_, d), F32)]
            + [pltpu.SemaphoreType.DMA] * 12
        ),
    )(src3, dst3, table, zeros_cd)


def _tc0(xp, w1p, bn, np_):
    fin = xp.shape[1]
    fo = w1p.shape[1]

    def body(x_ref, w_ref, h_ref):
        h_ref[...] = jnp.dot(x_ref[...], w_ref[...],
                             preferred_element_type=F32)

    return pl.pallas_call(
        body,
        grid=(np_ // bn,),
        in_specs=[
            pl.BlockSpec((bn, fin), lambda i: (i, 0)),
            pl.BlockSpec((fin, fo), lambda i: (0, 0)),
        ],
        out_specs=pl.BlockSpec((bn, fo), lambda i: (i, 0)),
        out_shape=jax.ShapeDtypeStruct((np_, fo), F32),
    )(xp, w1p)


def _tc1(h1, deg2, bn, np_):
    fo = h1.shape[1]
    half = fo // 2
    nb = np_ // bn

    def body(h_ref, da_ref, db_ref, hn_ref, di_ref):
        deg = da_ref[...] + db_ref[...] + 1.0
        dinv = lax.rsqrt(deg)
        h = h_ref[...]
        hn = h * dinv
        zpad = jnp.zeros((h.shape[0], DPAD1 - half), F32)
        hn_ref[0] = jnp.concatenate([hn[:, :half], zpad], axis=1)
        hn_ref[1] = jnp.concatenate([hn[:, half:], zpad], axis=1)
        di_ref[...] = dinv

    return pl.pallas_call(
        body,
        grid=(np_ // bn,),
        in_specs=[
            pl.BlockSpec((bn, fo), lambda i: (i, 0)),
            pl.BlockSpec((bn, 1), lambda i: (i, 0)),
            pl.BlockSpec((bn, 1), lambda i, nb=nb: (i + nb, 0)),
        ],
        out_specs=[
            pl.BlockSpec((NC, bn, DPAD1), lambda i: (0, i, 0)),
            pl.BlockSpec((bn, 1), lambda i: (i, 0)),
        ],
        out_shape=[
            jax.ShapeDtypeStruct((NC, np_, DPAD1), F32),
            jax.ShapeDtypeStruct((np_, 1), F32),
        ],
    )(h1, deg2, deg2)


def _tc2(acc, h1, dinv, b1p, w2, bn, np_):
    fo = h1.shape[1]
    half = fo // 2
    f3 = w2.shape[1]
    nb = np_ // bn

    def body(aa_ref, ab_ref, h_ref, di_ref, b_ref, w_ref, h3_ref, hn3_ref):
        dinv_c = di_ref[...]
        agg = jnp.concatenate([aa_ref[...][:, :half], ab_ref[...][:, :half]],
                              axis=1)
        out1 = dinv_c * agg + (dinv_c * dinv_c) * h_ref[...] + b_ref[...]
        out1 = _leaky(out1)
        h2 = jnp.maximum(out1[:, :half], out1[:, half:])
        h3 = jnp.dot(h2, w_ref[...], preferred_element_type=F32)
        h3_ref[...] = h3
        hn3 = h3 * dinv_c
        zpad = jnp.zeros((h3.shape[0], DPAD2 - f3), F32)
        hn3_ref[...] = jnp.concatenate([hn3, zpad], axis=1)

    return pl.pallas_call(
        body,
        grid=(np_ // bn,),
        in_specs=[
            pl.BlockSpec((bn, DPAD1), lambda i: (i, 0)),
            pl.BlockSpec((bn, DPAD1), lambda i, nb=nb: (i + nb, 0)),
            pl.BlockSpec((bn, fo), lambda i: (i, 0)),
            pl.BlockSpec((bn, 1), lambda i: (i, 0)),
            pl.BlockSpec((1, fo), lambda i: (0, 0)),
            pl.BlockSpec((half, f3), lambda i: (0, 0)),
        ],
        out_specs=[
            pl.BlockSpec((bn, f3), lambda i: (i, 0)),
            pl.BlockSpec((bn, DPAD2), lambda i: (i, 0)),
        ],
        out_shape=[
            jax.ShapeDtypeStruct((np_, f3), F32),
            jax.ShapeDtypeStruct((np_, DPAD2), F32),
        ],
    )(acc, acc, h1, dinv, b1p, w2)


def _tc3(acc, h3, dinv, b2, wl, bl, bn, np_, n):
    f3 = h3.shape[1]
    fl = wl.shape[1]
    nb = np_ // bn

    def body(aa_ref, ab_ref, h_ref, di_ref, b2_ref, w_ref, bl_ref, o_ref):
        dinv_c = di_ref[...]
        agg = aa_ref[...][:, :f3] + ab_ref[...][:, :f3]
        out2 = dinv_c * agg + (dinv_c * dinv_c) * h_ref[...] + b2_ref[...]
        out2 = _leaky(out2)
        o_ref[...] = (jnp.dot(out2, w_ref[...], preferred_element_type=F32)
                      + bl_ref[...])

    return pl.pallas_call(
        body,
        grid=(np_ // bn,),
        in_specs=[
            pl.BlockSpec((bn, DPAD2), lambda i: (i, 0)),
            pl.BlockSpec((bn, DPAD2), lambda i, nb=nb: (i + nb, 0)),
            pl.BlockSpec((bn, f3), lambda i: (i, 0)),
            pl.BlockSpec((bn, 1), lambda i: (i, 0)),
            pl.BlockSpec((1, f3), lambda i: (0, 0)),
            pl.BlockSpec((f3, fl), lambda i: (0, 0)),
            pl.BlockSpec((1, fl), lambda i: (0, 0)),
        ],
        out_specs=pl.BlockSpec((bn, fl), lambda i: (i, 0)),
        out_shape=jax.ShapeDtypeStruct((n, fl), F32),
    )(acc, acc, h3, dinv, b2, wl, bl)


def kernel(x, edge_index, W1, b1, W2, b2, Wl, bl):
    n = x.shape[0]
    e = edge_index.shape[1]
    fo = W1.shape[1]
    half = fo // 2
    f3 = W2.shape[1]

    rpt = -(-(n + 1) // (NS * 8)) * 8
    np_ = rpt * NS
    nc_ = -(-(-(-e // (NW * CH))) // U) * U
    ep = NW * CH * nc_

    perm = jnp.arange(fo).reshape(fo // 2, 2).T.reshape(fo)
    w1p = W1[:, perm]
    b1p = b1[perm][None, :]

    src_f = jnp.concatenate(
        [edge_index[0], jnp.full((ep - e,), n, jnp.int32)])
    dst_f = jnp.concatenate(
        [edge_index[1], jnp.full((ep - e,), n, jnp.int32)])
    src2 = src_f.reshape(NW, nc_, CH)
    dst2 = dst_f.reshape(NW, nc_, CH)
    src1 = src_f.reshape(NS, NC * nc_, CH)
    dst1 = dst_f.reshape(NS, NC * nc_, CH)
    xp = jnp.concatenate(
        [x, jnp.zeros((np_ - n, x.shape[1]), F32)], axis=0)

    cz = _row_chunk(rpt)
    zer_r = jnp.zeros((rpt,), F32)
    zer_ch = jnp.zeros((cz, DPAD1), F32)
    zer_cf = jnp.zeros((cz, DPAD2), F32)

    bn = np_ // 32

    h1 = _tc0(xp, w1p, bn, np_)
    deg = _sc_degree(dst2, zer_r, np_=np_, nc_=nc_)
    tab1, dinv = _tc1(h1, deg.reshape(NC * np_, 1), bn, np_)
    acc1 = _sc_aggregate(src1, dst1, tab1, zer_ch,
                         np_=np_, nc_=NC * nc_, d=DPAD1, colsplit=True)
    h3, hn3 = _tc2(acc1, h1, dinv, b1p, W2, bn, np_)
    acc2 = _sc_aggregate(src2, dst2, hn3, zer_cf,
                         np_=np_, nc_=nc_, d=DPAD2, colsplit=False)
    return _tc3(acc2, h3, dinv, b2[None, :], Wl, bl[None, :], bn, np_, n)
